# baseline (speedup 1.0000x reference)
import functools

import jax
import jax.numpy as jnp
from jax import lax
from jax.experimental import pallas as pl
from jax.experimental.pallas import tpu as pltpu
from jax.experimental.pallas import tpu_sc as plsc

NUM_CLASSES = 1000
FEAT = 64
BATCH = 16384
NC = 2
NS = 16
NW = NC * NS
BPW = BATCH // NW
GCH = 128
NG = BPW // GCH
CPAD = 1008


@functools.partial(
    pl.kernel,
    out_type=(
        jax.ShapeDtypeStruct((NW, CPAD), jnp.float32),
        jax.ShapeDtypeStruct((NW, CPAD), jnp.float32),
    ),
    mesh=plsc.VectorSubcoreMesh(core_axis_name="c", subcore_axis_name="s"),
    compiler_params=pltpu.CompilerParams(needs_layout_passes=False,
                                         use_tc_tiling_on_sc=False),
    scratch_types=[
        pltpu.VMEM((NG, GCH), jnp.int32),
        pltpu.VMEM((BPW, FEAT), jnp.float32),
        pltpu.VMEM((BPW, 16), jnp.float32),
        pltpu.VMEM((CPAD * 16,), jnp.float32),
        pltpu.VMEM((CPAD,), jnp.float32),
        pltpu.VMEM((CPAD,), jnp.float32),
        pltpu.SemaphoreType.DMA,
        pltpu.SemaphoreType.DMA,
    ],
)
def _sc_center_partials(feat_hbm, lab_hbm, negc_hbm, sums_out, cnt_out,
                        idx_v, feat_v, psum_v, sums_v, rsum_v, cnt_v,
                        sem_g, sem_f):
    wid = lax.axis_index("s") * NC + lax.axis_index("c")

    pltpu.sync_copy(lab_hbm.at[wid], idx_v)
    feat_cp = pltpu.async_copy(feat_hbm.at[wid], feat_v, sem_f)

    zeros16 = jnp.zeros((16,), jnp.float32)

    @plsc.parallel_loop(0, CPAD // 16, unroll=2)
    def _zero_sums(j):
        for u in range(16):
            sums_v[pl.ds(j * 256 + u * 16, 16)] = zeros16

    for u in range(CPAD // 16):
        cnt_v[pl.ds(u * 16, 16)] = zeros16

    feat_cp.wait()
    gathers = [
        pltpu.async_copy(negc_hbm.at[idx_v.at[g]],
                         feat_v.at[pl.ds(g * GCH, GCH)], sem_g, add=True)
        for g in range(NG)
    ]
    for cp in gathers:
        cp.wait()

    iota16 = lax.iota(jnp.int32, 16)
    ones16 = jnp.ones((16,), jnp.float32)

    @plsc.parallel_loop(0, BPW // 16, unroll=2)
    def _squares(kk):
        base = kk * 16
        for lane in range(16):
            i = base + lane
            d0 = feat_v[i, pl.ds(0, 16)]
            d1 = feat_v[i, pl.ds(16, 16)]
            d2 = feat_v[i, pl.ds(32, 16)]
            d3 = feat_v[i, pl.ds(48, 16)]
            psum_v[i] = (d0 * d0 + d1 * d1) + (d2 * d2 + d3 * d3)

    @plsc.parallel_loop(0, BPW // 16, unroll=2)
    def _accumulate(kk):
        base = kk * 16
        lab16 = idx_v[kk // 8, pl.ds((kk % 8) * 16, 16)]
        for lane in range(16):
            row_idx = lab16[lane] * 16 + iota16
            plsc.addupdate_scatter(sums_v, [row_idx], psum_v[base + lane])
            plsc.addupdate_scatter(cnt_v, [lab16], ones16,
                                   mask=iota16 == lane)

    iota_x16 = iota16 * 16

    @plsc.parallel_loop(0, CPAD // 16, unroll=2)
    def _rowsum(j):
        col0 = j * 256 + iota_x16
        tot = zeros16
        for c in range(16):
            tot = tot + plsc.load_gather(sums_v, [col0 + c])
        rsum_v[pl.ds(j * 16, 16)] = tot

    pltpu.sync_copy(rsum_v, sums_out.at[wid])
    pltpu.sync_copy(cnt_v, cnt_out.at[wid])


def _finish_body(sums_ref, cnt_ref, out_ref):
    s = jnp.sum(sums_ref[...], axis=0)
    n = jnp.sum(cnt_ref[...], axis=0)
    denom = jnp.maximum(n, 1.0) * FEAT
    per_class = jnp.where(n > 0, s / denom, 0.0)
    out_ref[...] = (jnp.sum(per_class) / BATCH).reshape(1, 1)


def kernel(features, labels, centers):
    feat_r = features.reshape(NW, BPW, FEAT)
    lab_r = labels.astype(jnp.int32).reshape(NW, NG, GCH)
    part_sums, part_cnt = _sc_center_partials(feat_r, lab_r, -centers)
    loss = pl.pallas_call(
        _finish_body,
        out_shape=jax.ShapeDtypeStruct((1, 1), jnp.float32),
    )(part_sums, part_cnt)
    return loss.reshape(())


# speedup vs baseline: 1.0220x; 1.0220x over previous
import functools

import jax
import jax.numpy as jnp
from jax import lax
from jax.experimental import pallas as pl
from jax.experimental.pallas import tpu as pltpu
from jax.experimental.pallas import tpu_sc as plsc

NUM_CLASSES = 1000
FEAT = 64
BATCH = 16384
NC = 2
NS = 16
NW = NC * NS
BPW = BATCH // NW
GCH = 128
NG = BPW // GCH
CPAD = 1008


@functools.partial(
    pl.kernel,
    out_type=(
        jax.ShapeDtypeStruct((NW, CPAD), jnp.float32),
        jax.ShapeDtypeStruct((NW, CPAD), jnp.float32),
    ),
    mesh=plsc.VectorSubcoreMesh(core_axis_name="c", subcore_axis_name="s"),
    compiler_params=pltpu.CompilerParams(needs_layout_passes=False,
                                         use_tc_tiling_on_sc=False),
    scratch_types=[
        pltpu.VMEM((NG, GCH), jnp.int32),
        pltpu.VMEM((BPW, FEAT), jnp.float32),
        pltpu.VMEM((BPW, 16), jnp.float32),
        pltpu.VMEM((CPAD * 16,), jnp.float32),
        pltpu.VMEM((CPAD,), jnp.float32),
        pltpu.VMEM((CPAD,), jnp.float32),
        pltpu.SemaphoreType.DMA,
        pltpu.SemaphoreType.DMA,
        pltpu.VMEM_SHARED((NUM_CLASSES, FEAT), jnp.float32),
    ],
)
def _sc_center_partials(feat_hbm, lab_hbm, negc_hbm, sums_out, cnt_out,
                        idx_v, feat_v, psum_v, sums_v, rsum_v, cnt_v,
                        sem_g, sem_f, negc_sh):
    sid = lax.axis_index("s")
    wid = sid * NC + lax.axis_index("c")

    @pl.when(sid == 0)
    def _stage_table():
        pltpu.sync_copy(negc_hbm, negc_sh)

    pltpu.sync_copy(lab_hbm.at[wid], idx_v)
    feat_cp = pltpu.async_copy(feat_hbm.at[wid], feat_v, sem_f)

    zeros16 = jnp.zeros((16,), jnp.float32)

    @plsc.parallel_loop(0, CPAD // 16, unroll=2)
    def _zero_sums(j):
        for u in range(16):
            sums_v[pl.ds(j * 256 + u * 16, 16)] = zeros16

    for u in range(CPAD // 16):
        cnt_v[pl.ds(u * 16, 16)] = zeros16

    plsc.subcore_barrier()
    feat_cp.wait()
    gathers = [
        pltpu.async_copy(negc_sh.at[idx_v.at[g]],
                         feat_v.at[pl.ds(g * GCH, GCH)], sem_g, add=True)
        for g in range(NG)
    ]
    for cp in gathers:
        cp.wait()

    iota16 = lax.iota(jnp.int32, 16)
    ones16 = jnp.ones((16,), jnp.float32)

    @plsc.parallel_loop(0, BPW // 16, unroll=2)
    def _squares(kk):
        base = kk * 16
        for lane in range(16):
            i = base + lane
            d0 = feat_v[i, pl.ds(0, 16)]
            d1 = feat_v[i, pl.ds(16, 16)]
            d2 = feat_v[i, pl.ds(32, 16)]
            d3 = feat_v[i, pl.ds(48, 16)]
            psum_v[i] = (d0 * d0 + d1 * d1) + (d2 * d2 + d3 * d3)

    @plsc.parallel_loop(0, BPW // 16, unroll=2)
    def _accumulate(kk):
        base = kk * 16
        lab16 = idx_v[kk // 8, pl.ds((kk % 8) * 16, 16)]
        for lane in range(16):
            row_idx = lab16[lane] * 16 + iota16
            plsc.addupdate_scatter(sums_v, [row_idx], psum_v[base + lane])
            plsc.addupdate_scatter(cnt_v, [lab16], ones16,
                                   mask=iota16 == lane)

    iota_x16 = iota16 * 16

    @plsc.parallel_loop(0, CPAD // 16, unroll=2)
    def _rowsum(j):
        col0 = j * 256 + iota_x16
        tot = zeros16
        for c in range(16):
            tot = tot + plsc.load_gather(sums_v, [col0 + c])
        rsum_v[pl.ds(j * 16, 16)] = tot

    pltpu.sync_copy(rsum_v, sums_out.at[wid])
    pltpu.sync_copy(cnt_v, cnt_out.at[wid])


def _finish_body(sums_ref, cnt_ref, out_ref):
    s = jnp.sum(sums_ref[...], axis=0)
    n = jnp.sum(cnt_ref[...], axis=0)
    denom = jnp.maximum(n, 1.0) * FEAT
    per_class = jnp.where(n > 0, s / denom, 0.0)
    out_ref[...] = (jnp.sum(per_class) / BATCH).reshape(1, 1)


def kernel(features, labels, centers):
    feat_r = features.reshape(NW, BPW, FEAT)
    lab_r = labels.astype(jnp.int32).reshape(NW, NG, GCH)
    part_sums, part_cnt = _sc_center_partials(feat_r, lab_r, -centers)
    loss = pl.pallas_call(
        _finish_body,
        out_shape=jax.ShapeDtypeStruct((1, 1), jnp.float32),
    )(part_sums, part_cnt)
    return loss.reshape(())


# speedup vs baseline: 1.1307x; 1.1063x over previous
import functools

import jax
import jax.numpy as jnp
from jax import lax
from jax.experimental import pallas as pl
from jax.experimental.pallas import tpu as pltpu
from jax.experimental.pallas import tpu_sc as plsc

NUM_CLASSES = 1000
FEAT = 64
BATCH = 16384
NC = 2
NS = 16
NW = NC * NS
BPW = BATCH // NW
GCH = 128
NG = BPW // GCH
CPAD = 1008
CNT_OFF = 1024
ACC = 2048


@functools.partial(
    pl.kernel,
    out_type=jax.ShapeDtypeStruct((NW, ACC), jnp.float32),
    mesh=plsc.VectorSubcoreMesh(core_axis_name="c", subcore_axis_name="s"),
    compiler_params=pltpu.CompilerParams(needs_layout_passes=False,
                                         use_tc_tiling_on_sc=False),
    scratch_types=[
        pltpu.VMEM((NG, GCH), jnp.int32),
        pltpu.VMEM((BPW, FEAT), jnp.float32),
        pltpu.VMEM((BPW * 16,), jnp.float32),
        pltpu.VMEM((ACC,), jnp.float32),
        pltpu.SemaphoreType.DMA,
        pltpu.SemaphoreType.DMA,
        pltpu.VMEM_SHARED((NUM_CLASSES, FEAT), jnp.float32),
    ],
)
def _sc_center_partials(feat_hbm, lab_hbm, negc_hbm, acc_out,
                        idx_v, feat_v, psum_v, acc_v, sem_g, sem_f, negc_sh):
    sid = lax.axis_index("s")
    wid = sid * NC + lax.axis_index("c")

    @pl.when(sid == 0)
    def _stage_table():
        pltpu.sync_copy(negc_hbm, negc_sh)

    pltpu.sync_copy(lab_hbm.at[wid], idx_v)
    feat_cp = pltpu.async_copy(feat_hbm.at[wid], feat_v, sem_f)

    zeros16 = jnp.zeros((16,), jnp.float32)

    @plsc.parallel_loop(0, ACC // 256, unroll=2)
    def _zero_acc(j):
        for u in range(16):
            acc_v[pl.ds(j * 256 + u * 16, 16)] = zeros16

    plsc.subcore_barrier()
    feat_cp.wait()
    gathers = [
        pltpu.async_copy(negc_sh.at[idx_v.at[g]],
                         feat_v.at[pl.ds(g * GCH, GCH)], sem_g, add=True)
        for g in range(NG)
    ]
    for cp in gathers:
        cp.wait()

    iota16 = lax.iota(jnp.int32, 16)
    ones16 = jnp.ones((16,), jnp.float32)
    iota_x16 = iota16 * 16

    @plsc.parallel_loop(0, BPW // 16, unroll=2)
    def _squares(kk):
        base = kk * 16
        for lane in range(16):
            i = base + lane
            d0 = feat_v[i, pl.ds(0, 16)]
            d1 = feat_v[i, pl.ds(16, 16)]
            d2 = feat_v[i, pl.ds(32, 16)]
            d3 = feat_v[i, pl.ds(48, 16)]
            psum_v[pl.ds(i * 16, 16)] = (d0 * d0 + d1 * d1) + (d2 * d2 + d3 * d3)

    @plsc.parallel_loop(0, BPW // 16, unroll=2)
    def _accumulate(kk):
        base16 = kk * 256 + iota_x16
        lab16 = idx_v[kk // 8, pl.ds((kk % 8) * 16, 16)]
        labc16 = lab16 + CNT_OFF
        t0 = zeros16
        t1 = zeros16
        t2 = zeros16
        t3 = zeros16
        for c in range(0, 16, 4):
            t0 = t0 + plsc.load_gather(psum_v, [base16 + c])
            t1 = t1 + plsc.load_gather(psum_v, [base16 + (c + 1)])
            t2 = t2 + plsc.load_gather(psum_v, [base16 + (c + 2)])
            t3 = t3 + plsc.load_gather(psum_v, [base16 + (c + 3)])
        tot = (t0 + t1) + (t2 + t3)
        for lane in range(16):
            m = iota16 == lane
            plsc.addupdate_scatter(acc_v, [lab16], tot, mask=m)
            plsc.addupdate_scatter(acc_v, [labc16], ones16, mask=m)

    pltpu.sync_copy(acc_v, acc_out.at[wid])


def _finish_body(acc_ref, out_ref):
    tot = jnp.sum(acc_ref[...], axis=0)
    s = tot[:CPAD]
    n = tot[CNT_OFF:CNT_OFF + CPAD]
    denom = jnp.maximum(n, 1.0) * FEAT
    per_class = jnp.where(n > 0, s / denom, 0.0)
    out_ref[...] = (jnp.sum(per_class) / BATCH).reshape(1, 1)


def kernel(features, labels, centers):
    feat_r = features.reshape(NW, BPW, FEAT)
    lab_r = labels.astype(jnp.int32).reshape(NW, NG, GCH)
    part_acc = _sc_center_partials(feat_r, lab_r, -centers)
    loss = pl.pallas_call(
        _finish_body,
        out_shape=jax.ShapeDtypeStruct((1, 1), jnp.float32),
    )(part_acc)
    return loss.reshape(())
